# Initial kernel scaffold; baseline (speedup 1.0000x reference)
import jax
import jax.numpy as jnp
from jax.experimental import pallas as pl


def kernel(feat, segment_ids):
    raise NotImplementedError("write your pallas kernel here")


# speedup vs baseline: 1.1296x; 1.1296x over previous
import functools

import jax
import jax.numpy as jnp
from jax import lax
from jax.experimental import pallas as pl
from jax.experimental.pallas import tpu as pltpu
from jax.experimental.pallas import tpu_sc as plsc

_N = 100000
_D = 512
_S = 50000
_NC = 2
_NS = 16
_M = 32
_K = 32


def _sc_body(n, d, s_out, m, k, nw, rpw,
             feat_hbm, ids_hbm, out_hbm, feat_v, ids_v, idb_lo, idb_hi,
             idb_bs, out_v, acc_v):
    dv = d // 16
    cid = lax.axis_index("c")
    sid_ax = lax.axis_index("s")
    wid = sid_ax * _NC + cid
    lo = wid * rpw
    hi = lo + rpw

    def pair_at(buf, pos):
        a = (pos - 1) // 8 * 8
        pltpu.sync_copy(ids_hbm.at[pl.ds(a, 16)], buf.at[pl.ds(0, 16)])
        pair = buf[pl.ds(pos - 1 - a, 16)]
        return pair[0], pair[1]

    prev_l, cur_l = pair_at(idb_lo, jnp.maximum(lo, 1))
    p_lo = jnp.where(wid == 0, 0,
                     jnp.where(prev_l == cur_l, cur_l + 1, cur_l))

    prev_h, cur_h = pair_at(idb_hi, jnp.minimum(hi, n - 8))
    p_hi = jnp.where(wid == nw - 1, s_out,
                     jnp.where(prev_h == cur_h, cur_h + 1, cur_h))

    hb = jnp.minimum(hi // 8 * 8, n - 16)
    pltpu.sync_copy(ids_hbm.at[pl.ds(hb, 16)], idb_bs.at[pl.ds(0, 16)])
    win = idb_bs[pl.ds(0, 16)]
    in_tail = jnp.int32(0)
    for lane in range(16):
        active = jnp.logical_and(hb + lane >= hi, win[lane] < p_hi)
        in_tail = in_tail + jnp.where(active, 1, 0)
    nlanes = jnp.maximum(16 - (hi - hb), 0)
    fast_hit = jnp.logical_or(in_tail < nlanes, hi >= n)
    r_end_fast = jnp.where(hi >= n, n, hi + in_tail)

    def bs_step(_, st):
        lo_b, hi_b = st
        mid = (lo_b + hi_b) // 2
        nonempty = mid < hi_b
        safe = jnp.minimum(mid, n - 1)
        a = jnp.minimum(safe // 8 * 8, n - 16)
        pltpu.sync_copy(ids_hbm.at[pl.ds(a, 16)], idb_bs.at[pl.ds(0, 16)])
        v = idb_bs[pl.ds(safe - a, 16)][0]
        go_right = jnp.logical_and(nonempty, v < p_hi)
        lo_b2 = jnp.where(go_right, mid + 1, lo_b)
        hi_b2 = jnp.where(jnp.logical_and(nonempty,
                                          jnp.logical_not(go_right)),
                          mid, hi_b)
        return (lo_b2, hi_b2)

    nbs = jnp.where(fast_hit, 0, 17)
    lo_bs, _ = lax.fori_loop(0, nbs, bs_step, (hb + 16, jnp.int32(n)))
    r_end = jnp.where(fast_hit, r_end_fast, lo_bs)

    zvec = jnp.zeros((16,), jnp.float32)

    def maybe_flush(nxt_after):
        emitted = nxt_after - p_lo

        @pl.when(emitted % k == 0)
        def _():
            pltpu.sync_copy(out_v, out_hbm.at[pl.ds(nxt_after - k, k)])

    def emit_zeros_until(nxt, tgt):
        def zbody(_, nxt0):
            loc = (nxt0 - p_lo) % k
            for j in range(dv):
                out_v[loc, pl.ds(j * 16, 16)] = zvec
            maybe_flush(nxt0 + 1)
            return nxt0 + 1

        return lax.fori_loop(0, tgt - nxt, zbody, nxt)

    def finalize(cur, cnt, nxt):
        def do(nxt0):
            nxt1 = emit_zeros_until(nxt0, cur)
            scale = jnp.full((16,), 1.0, jnp.float32) / jnp.full(
                (16,), cnt, jnp.float32)
            loc = (nxt1 - p_lo) % k
            for j in range(dv):
                out_v[loc, pl.ds(j * 16, 16)] = acc_v[j] * scale
            maybe_flush(nxt1 + 1)
            return nxt1 + 1

        return lax.cond(cur >= p_lo, do, lambda v: v, nxt)

    def bbody(t, st):
        cb, cur, cnt, nxt = st
        r = lo + t
        newcb = r // m * m

        @pl.when(newcb != cb)
        def _():
            pltpu.sync_copy(feat_hbm.at[pl.ds(newcb, m)], feat_v)
            pltpu.sync_copy(ids_hbm.at[pl.ds(newcb, m)],
                            ids_v.at[pl.ds(0, m)])

        i = r - newcb
        sid = ids_v[pl.ds(i, 16)][0]
        fin = sid != cur

        def dofin(v):
            v1 = finalize(cur, cnt, v)
            for j in range(dv):
                acc_v[j] = zvec
            return v1

        nxt1 = lax.cond(fin, dofin, lambda v: v, nxt)
        cnt1 = jnp.where(fin, 0.0, cnt)
        for j in range(dv):
            acc_v[j] = acc_v[j] + feat_v[i, pl.ds(j * 16, 16)]
        return (newcb, sid, cnt1 + 1.0, nxt1)

    init = (jnp.int32(-1), jnp.int32(-1), jnp.float32(0.0), p_lo)
    _, cur_f, cnt_f, nxt_f = lax.fori_loop(0, r_end - lo, bbody, init)

    nxt_f = finalize(cur_f, cnt_f, nxt_f)
    nxt_f = emit_zeros_until(nxt_f, p_hi)
    rem = (nxt_f - p_lo) % k
    base = nxt_f - rem

    def fbody(j, carry):
        pltpu.sync_copy(out_v.at[pl.ds(j, 1)],
                        out_hbm.at[pl.ds(base + j, 1)])
        return carry

    lax.fori_loop(0, rem, fbody, jnp.int32(0))


def _make_sc_call(n, d, s_out, m, k):
    nw = _NC * _NS
    rpw = n // nw
    assert n % nw == 0 and n % m == 0 and d % 16 == 0

    mesh = plsc.VectorSubcoreMesh(core_axis_name="c", subcore_axis_name="s",
                                  num_cores=_NC, num_subcores=_NS)
    return pl.kernel(
        functools.partial(_sc_body, n, d, s_out, m, k, nw, rpw),
        out_type=jax.ShapeDtypeStruct((s_out, d), jnp.float32),
        mesh=mesh,
        compiler_params=pltpu.CompilerParams(use_tc_tiling_on_sc=False),
        scratch_types=[
            pltpu.VMEM((m, d), jnp.float32),
            pltpu.VMEM((m + 16,), jnp.int32),
            pltpu.VMEM((32,), jnp.int32),
            pltpu.VMEM((32,), jnp.int32),
            pltpu.VMEM((32,), jnp.int32),
            pltpu.VMEM((k, d), jnp.float32),
            pltpu.VMEM((d // 16, 16), jnp.float32),
        ],
    )


@jax.jit
def kernel(feat, segment_ids):
    ids32 = segment_ids.astype(jnp.int32)
    call = _make_sc_call(_N, _D, _S, _M, _K)
    return call(feat, ids32)


# speedup vs baseline: 2.9024x; 2.5694x over previous
import functools

import jax
import jax.numpy as jnp
from jax import lax
from jax.experimental import pallas as pl
from jax.experimental.pallas import tpu as pltpu
from jax.experimental.pallas import tpu_sc as plsc

_N = 100000
_D = 512
_S = 50000
_NC = 2
_NS = 16
_M = 64
_K = 64


def _sc_body(n, d, s_out, m, k, nw, rpw,
             feat_hbm, ids_hbm, out_hbm, feat_v, ids_v,
             idb_lo, idb_hi, idb_bs, out_v, sf0, sf1, si0, si1):
    dv = d // 16
    cid = lax.axis_index("c")
    sid_ax = lax.axis_index("s")
    wid = sid_ax * _NC + cid
    lo = wid * rpw
    hi = lo + rpw

    def pair_at(buf, pos):
        a = (pos - 1) // 8 * 8
        pltpu.sync_copy(ids_hbm.at[pl.ds(a, 16)], buf.at[pl.ds(0, 16)])
        pair = buf[pl.ds(pos - 1 - a, 16)]
        return pair[0], pair[1]

    prev_l, cur_l = pair_at(idb_lo, jnp.maximum(lo, 1))
    p_lo = jnp.where(wid == 0, 0,
                     jnp.where(prev_l == cur_l, cur_l + 1, cur_l))

    prev_h, cur_h = pair_at(idb_hi, jnp.minimum(hi, n - 8))
    p_hi = jnp.where(wid == nw - 1, s_out,
                     jnp.where(prev_h == cur_h, cur_h + 1, cur_h))

    hb = jnp.minimum(hi // 8 * 8, n - 16)
    pltpu.sync_copy(ids_hbm.at[pl.ds(hb, 16)], idb_bs.at[pl.ds(0, 16)])
    win = idb_bs[pl.ds(0, 16)]
    in_tail = jnp.int32(0)
    for lane in range(16):
        active = jnp.logical_and(hb + lane >= hi, win[lane] < p_hi)
        in_tail = in_tail + jnp.where(active, 1, 0)
    nlanes = jnp.maximum(16 - (hi - hb), 0)
    fast_hit = jnp.logical_or(in_tail < nlanes, hi >= n)
    r_end_fast = jnp.where(hi >= n, n, hi + in_tail)

    def bs_step(_, st):
        lo_b, hi_b = st
        mid = (lo_b + hi_b) // 2
        nonempty = mid < hi_b
        safe = jnp.minimum(mid, n - 1)
        a = jnp.minimum(safe // 8 * 8, n - 16)
        pltpu.sync_copy(ids_hbm.at[pl.ds(a, 16)], idb_bs.at[pl.ds(0, 16)])
        v = idb_bs[pl.ds(safe - a, 16)][0]
        go_right = jnp.logical_and(nonempty, v < p_hi)
        lo_b2 = jnp.where(go_right, mid + 1, lo_b)
        hi_b2 = jnp.where(jnp.logical_and(nonempty,
                                          jnp.logical_not(go_right)),
                          mid, hi_b)
        return (lo_b2, hi_b2)

    nbs = jnp.where(fast_hit, 0, 17)
    lo_bs, _ = lax.fori_loop(0, nbs, bs_step, (hb + 16, jnp.int32(n)))
    r_end = jnp.where(fast_hit, r_end_fast, lo_bs)

    zvec = jnp.zeros((16,), jnp.float32)

    def maybe_flush(nxt_after):
        emitted = nxt_after - p_lo

        @pl.when(emitted % k == 0)
        def _():
            pltpu.sync_copy(out_v,
                            out_hbm.at[pl.ds((nxt_after - k) * d, k * d)])

    def emit_zeros_until(nxt, tgt):
        def zbody(_, nxt0):
            loc = (nxt0 - p_lo) % k
            for j in range(dv):
                out_v[pl.ds(loc * d + j * 16, 16)] = zvec
            maybe_flush(nxt0 + 1)
            return nxt0 + 1

        return lax.fori_loop(0, tgt - nxt, zbody, nxt)

    def finalize(cur, cnt, nxt, acc):
        def do(nxt0):
            nxt1 = emit_zeros_until(nxt0, cur)
            scale = jnp.full((16,), 1.0, jnp.float32) / jnp.full(
                (16,), cnt, jnp.float32)
            loc = (nxt1 - p_lo) % k
            for j in range(dv):
                out_v[pl.ds(loc * d + j * 16, 16)] = acc[j] * scale
            maybe_flush(nxt1 + 1)
            return nxt1 + 1

        return lax.cond(cur >= p_lo, do, lambda v: v, nxt)

    c0 = lo // m
    c1 = (r_end + m - 1) // m
    islot = m + 16

    def start_fetch(c, fslot, ioff, semf, semi):
        pltpu.async_copy(feat_hbm.at[pl.ds(c * m * d, m * d)],
                         feat_v.at[pl.ds(fslot * m * d, m * d)], semf)
        pltpu.async_copy(ids_hbm.at[pl.ds(c * m, m)],
                         ids_v.at[pl.ds(ioff, m)], semi)

    def wait_fetch(c, fslot, ioff, semf, semi):
        pltpu.make_async_copy(feat_hbm.at[pl.ds(c * m * d, m * d)],
                              feat_v.at[pl.ds(fslot * m * d, m * d)],
                              semf).wait()
        pltpu.make_async_copy(ids_hbm.at[pl.ds(c * m, m)],
                              ids_v.at[pl.ds(ioff, m)], semi).wait()

    start_fetch(c0, 0, 0, sf0, si0)

    def inner(c, par, st):
        base = c * m
        start_row = jnp.maximum(lo, base)
        end_row = jnp.minimum(r_end, base + m)
        foff = par * m
        ioff = par * islot

        def rbody(t, rst):
            cur, cnt, nxt = rst[0], rst[1], rst[2]
            acc = rst[3:]
            i = start_row - base + t
            sid = ids_v[pl.ds(ioff + i, 16)][0]
            fin = sid != cur

            def dofin(v):
                return finalize(cur, cnt, v, acc)

            nxt1 = lax.cond(fin, dofin, lambda v: v, nxt)
            cnt1 = jnp.where(fin, 0.0, cnt)
            keep = jnp.full((16,), jnp.where(fin, 0.0, 1.0), jnp.float32)
            rb = (foff + i) * d
            acc2 = tuple(acc[j] * keep + feat_v[pl.ds(rb + j * 16, 16)]
                         for j in range(dv))
            return (sid, cnt1 + 1.0, nxt1) + acc2

        return lax.fori_loop(0, end_row - start_row, rbody, st)

    def cbody(t, st):
        c = c0 + t
        par = t % 2

        @pl.when(par == 0)
        def _():
            wait_fetch(c, 0, 0, sf0, si0)

        @pl.when(par == 1)
        def _():
            wait_fetch(c, 1, islot, sf1, si1)

        @pl.when(jnp.logical_and(par == 0, c + 1 < c1))
        def _():
            start_fetch(c + 1, 1, islot, sf1, si1)

        @pl.when(jnp.logical_and(par == 1, c + 1 < c1))
        def _():
            start_fetch(c + 1, 0, 0, sf0, si0)

        return inner(c, par, st)

    init = (jnp.int32(-1), jnp.float32(0.0), p_lo) + (zvec,) * dv
    fst = lax.fori_loop(0, c1 - c0, cbody, init)
    cur_f, cnt_f, nxt_f = fst[0], fst[1], fst[2]
    acc_f = fst[3:]

    nxt_f = finalize(cur_f, cnt_f, nxt_f, acc_f)
    nxt_f = emit_zeros_until(nxt_f, p_hi)
    rem = (nxt_f - p_lo) % k
    base = nxt_f - rem

    def fbody(j, carry):
        pltpu.sync_copy(out_v.at[pl.ds(j * d, d)],
                        out_hbm.at[pl.ds((base + j) * d, d)])
        return carry

    lax.fori_loop(0, rem, fbody, jnp.int32(0))


def _make_sc_call(n, d, s_out, m, k):
    nw = _NC * _NS
    rpw = n // nw
    assert n % nw == 0 and n % m == 0 and d % 16 == 0

    mesh = plsc.VectorSubcoreMesh(core_axis_name="c", subcore_axis_name="s",
                                  num_cores=_NC, num_subcores=_NS)
    return pl.kernel(
        functools.partial(_sc_body, n, d, s_out, m, k, nw, rpw),
        out_type=jax.ShapeDtypeStruct((s_out * d,), jnp.float32),
        mesh=mesh,
        compiler_params=pltpu.CompilerParams(use_tc_tiling_on_sc=False),
        scratch_types=[
            pltpu.VMEM((2 * m * d,), jnp.float32),
            pltpu.VMEM((2 * (m + 16),), jnp.int32),
            pltpu.VMEM((32,), jnp.int32),
            pltpu.VMEM((32,), jnp.int32),
            pltpu.VMEM((32,), jnp.int32),
            pltpu.VMEM((k * d,), jnp.float32),
            pltpu.SemaphoreType.DMA,
            pltpu.SemaphoreType.DMA,
            pltpu.SemaphoreType.DMA,
            pltpu.SemaphoreType.DMA,
        ],
    )


@jax.jit
def kernel(feat, segment_ids):
    ids32 = segment_ids.astype(jnp.int32)
    call = _make_sc_call(_N, _D, _S, _M, _K)
    return call(feat.reshape(_N * _D), ids32).reshape(_S, _D)


# speedup vs baseline: 4.5305x; 1.5610x over previous
import functools

import jax
import jax.numpy as jnp
from jax import lax
from jax.experimental import pallas as pl
from jax.experimental.pallas import tpu as pltpu
from jax.experimental.pallas import tpu_sc as plsc

_N = 100000
_D = 512
_S = 50000
_NC = 2
_NS = 16
_M = 80
_K = 64


def _sc_body(n, d, s_out, m, k, nw, rpw,
             feat_hbm, ids_hbm, ids8_hbm, out_hbm, feat_v, ids_v,
             idb_lo, idb_hi, idb_bs, out_v, sf0, sf1, si0, si1):
    dv = d // 16
    cid = lax.axis_index("c")
    sid_ax = lax.axis_index("s")
    wid = sid_ax * _NC + cid
    lo = wid * rpw
    hi = lo + rpw

    def pair_at(buf, pos):
        a = pl.multiple_of((pos - 1) * 8, 8)
        pltpu.sync_copy(ids8_hbm.at[pl.ds(a, 16)], buf.at[pl.ds(0, 16)])
        pair = buf[pl.ds(0, 16)]
        return pair[0], pair[8]

    prev_l, cur_l = pair_at(idb_lo, jnp.maximum(lo, 1))
    p_lo = jnp.where(wid == 0, 0,
                     jnp.where(prev_l == cur_l, cur_l + 1, cur_l))

    prev_h, cur_h = pair_at(idb_hi, jnp.minimum(hi, n - 8))
    p_hi = jnp.where(wid == nw - 1, s_out,
                     jnp.where(prev_h == cur_h, cur_h + 1, cur_h))

    p_lo = pl.multiple_of(p_lo // 8 * 8, 8)
    p_hi = pl.multiple_of(jnp.minimum((p_hi + 7) // 8 * 8, s_out), 8)

    def count_lt(buf, target):
        cnt = jnp.int32(0)
        for q in range(4):
            v = buf[pl.ds(q * 16, 16)]
            cnt = cnt + plsc.all_reduce_population_count(v < target)[0]
        return cnt

    def lower_bound(target, lo0):
        def bs_step(_, st):
            lo_b, hi_b = st
            mid = (lo_b + hi_b) // 2
            nonempty = mid < hi_b
            safe = jnp.minimum(mid, n - 1)
            a = pl.multiple_of(safe * 8, 8)
            pltpu.sync_copy(ids8_hbm.at[pl.ds(a, 16)],
                            idb_bs.at[pl.ds(0, 16)])
            v = idb_bs[pl.ds(0, 16)][0]
            go_right = jnp.logical_and(nonempty, v < target)
            lo_b2 = jnp.where(go_right, mid + 1, lo_b)
            hi_b2 = jnp.where(jnp.logical_and(nonempty,
                                              jnp.logical_not(go_right)),
                              mid, hi_b)
            return (lo_b2, hi_b2)

        res, _ = lax.fori_loop(0, 17, bs_step, (lo0, jnp.int32(n)))
        return res

    wa = pl.multiple_of(jnp.maximum((lo // 8 * 8) - 48, 0), 8)
    pltpu.sync_copy(ids_hbm.at[pl.ds(wa, 64)], idb_lo.at[pl.ds(0, 64)])
    clt_a = count_lt(idb_lo, p_lo)
    head_lt = idb_lo[pl.ds(0, 16)][0] < p_lo
    fast_b = jnp.logical_or(
        wid == 0,
        jnp.logical_and(jnp.logical_or(head_lt, wa == 0), clt_a < 64))
    r_begin_fast = jnp.where(wid == 0, 0, wa + clt_a)

    wb = pl.multiple_of(jnp.minimum(hi // 8 * 8, n - 64), 8)
    pltpu.sync_copy(ids_hbm.at[pl.ds(wb, 64)], idb_hi.at[pl.ds(0, 64)])
    clt_b = count_lt(idb_hi, p_hi)
    fast_e = jnp.logical_or(wid == nw - 1, clt_b < 64)
    r_end_fast = jnp.where(wid == nw - 1, n, wb + clt_b)

    nbs_b = jnp.where(fast_b, 0, 1)
    r_begin = lax.fori_loop(
        0, nbs_b, lambda _, rb: lower_bound(p_lo, jnp.int32(0)),
        r_begin_fast)
    nbs_e = jnp.where(fast_e, 0, 1)
    r_end = lax.fori_loop(
        0, nbs_e, lambda _, re_: lower_bound(p_hi, hi), r_end_fast)

    zvec = jnp.zeros((16,), jnp.float32)

    def maybe_flush(nxt_after):
        emitted = nxt_after - p_lo

        @pl.when(emitted % k == 0)
        def _():
            row0 = pl.multiple_of(nxt_after - k, 8)
            pltpu.sync_copy(out_v, out_hbm.at[pl.ds(row0, k)])

    def emit_zeros_until(nxt, tgt):
        def zbody(_, nxt0):
            loc = (nxt0 - p_lo) % k
            for j in range(dv):
                out_v[loc, pl.ds(j * 16, 16)] = zvec
            maybe_flush(nxt0 + 1)
            return nxt0 + 1

        return lax.fori_loop(0, tgt - nxt, zbody, nxt)

    def finalize(cur, cnt, nxt, acc):
        def do(nxt0):
            nxt1 = emit_zeros_until(nxt0, cur)
            scale = jnp.full((16,), 1.0, jnp.float32) / jnp.full(
                (16,), cnt, jnp.float32)
            loc = (nxt1 - p_lo) % k
            for j in range(dv):
                out_v[loc, pl.ds(j * 16, 16)] = acc[j] * scale
            maybe_flush(nxt1 + 1)
            return nxt1 + 1

        return lax.cond(cur >= p_lo, do, lambda v: v, nxt)

    c0 = r_begin // m
    c1 = (r_end + m - 1) // m
    islot = m * 8 + 16

    def start_fetch(c, fslot, ioff, semf, semi):
        pltpu.async_copy(feat_hbm.at[pl.ds(pl.multiple_of(c * m, 8), m)],
                         feat_v.at[pl.ds(fslot * m, m)], semf)
        pltpu.async_copy(ids8_hbm.at[pl.ds(c * m * 8, m * 8)],
                         ids_v.at[pl.ds(ioff, m * 8)], semi)

    def wait_fetch(c, fslot, ioff, semf, semi):
        pltpu.make_async_copy(feat_hbm.at[pl.ds(pl.multiple_of(c * m, 8), m)],
                              feat_v.at[pl.ds(fslot * m, m)],
                              semf).wait()
        pltpu.make_async_copy(ids8_hbm.at[pl.ds(c * m * 8, m * 8)],
                              ids_v.at[pl.ds(ioff, m * 8)], semi).wait()

    @pl.when(c1 > c0)
    def _():
        start_fetch(c0, 0, 0, sf0, si0)

    def inner(c, par, st):
        base = c * m
        start_row = jnp.maximum(r_begin, base)
        end_row = jnp.minimum(r_end, base + m)
        foff = par * m
        ioff = par * islot

        def rbody(t, rst):
            cur, cnt, nxt = rst[0], rst[1], rst[2]
            acc = rst[3:]
            i = start_row - base + t
            sid = ids_v[pl.ds(pl.multiple_of(ioff + i * 8, 8), 16)][0]
            fin = sid != cur

            def dofin(v):
                return finalize(cur, cnt, v, acc)

            nxt1 = lax.cond(fin, dofin, lambda v: v, nxt)
            cnt1 = jnp.where(fin, 0.0, cnt)
            keep = jnp.full((16,), jnp.where(fin, 0.0, 1.0), jnp.float32)
            acc2 = tuple(acc[j] * keep + feat_v[foff + i, pl.ds(j * 16, 16)]
                         for j in range(dv))
            return (sid, cnt1 + 1.0, nxt1) + acc2

        return lax.fori_loop(0, end_row - start_row, rbody, st)

    def cbody(t, st):
        c = c0 + t
        par = t % 2

        @pl.when(par == 0)
        def _():
            wait_fetch(c, 0, 0, sf0, si0)

        @pl.when(par == 1)
        def _():
            wait_fetch(c, 1, islot, sf1, si1)

        @pl.when(jnp.logical_and(par == 0, c + 1 < c1))
        def _():
            start_fetch(c + 1, 1, islot, sf1, si1)

        @pl.when(jnp.logical_and(par == 1, c + 1 < c1))
        def _():
            start_fetch(c + 1, 0, 0, sf0, si0)

        return inner(c, par, st)

    init = (jnp.int32(-1), jnp.float32(0.0), p_lo) + (zvec,) * dv
    fst = lax.fori_loop(0, c1 - c0, cbody, init)
    cur_f, cnt_f, nxt_f = fst[0], fst[1], fst[2]
    acc_f = fst[3:]

    nxt_f = finalize(cur_f, cnt_f, nxt_f, acc_f)
    nxt_f = emit_zeros_until(nxt_f, p_hi)
    rem = (nxt_f - p_lo) % k
    fbase = nxt_f - rem

    fbase = pl.multiple_of(fbase, 8)

    def fbody(q, carry):
        pltpu.sync_copy(out_v.at[pl.ds(q * 8, 8)],
                        out_hbm.at[pl.ds(fbase + q * 8, 8)])
        return carry

    lax.fori_loop(0, rem // 8, fbody, jnp.int32(0))


def _make_sc_call(n, d, s_out, m, k):
    nw = _NC * _NS
    rpw = n // nw
    assert n % nw == 0 and n % m == 0 and d % 16 == 0
    assert m % 8 == 0 and k % 8 == 0 and s_out % 8 == 0

    mesh = plsc.VectorSubcoreMesh(core_axis_name="c", subcore_axis_name="s",
                                  num_cores=_NC, num_subcores=_NS)
    return pl.kernel(
        functools.partial(_sc_body, n, d, s_out, m, k, nw, rpw),
        out_type=jax.ShapeDtypeStruct((s_out, d), jnp.float32),
        mesh=mesh,
        compiler_params=pltpu.CompilerParams(use_tc_tiling_on_sc=True,
                                             needs_layout_passes=False),
        scratch_types=[
            pltpu.VMEM((2 * m, d), jnp.float32),
            pltpu.VMEM((2 * (m * 8 + 16),), jnp.int32),
            pltpu.VMEM((80,), jnp.int32),
            pltpu.VMEM((80,), jnp.int32),
            pltpu.VMEM((32,), jnp.int32),
            pltpu.VMEM((k, d), jnp.float32),
            pltpu.SemaphoreType.DMA,
            pltpu.SemaphoreType.DMA,
            pltpu.SemaphoreType.DMA,
            pltpu.SemaphoreType.DMA,
        ],
    )


@jax.jit
def kernel(feat, segment_ids):
    ids32 = segment_ids.astype(jnp.int32)
    ids8 = jnp.pad(jnp.repeat(ids32, 8), (0, 16))
    call = _make_sc_call(_N, _D, _S, _M, _K)
    return call(feat, ids32, ids8)
